# Initial kernel scaffold; baseline (speedup 1.0000x reference)
import jax
import jax.numpy as jnp
from jax.experimental import pallas as pl


def kernel(x, edge_index, W1, b1, W2, b2, W3, b3, Wfc, bfc):
    raise NotImplementedError("write your pallas kernel here")


# speedup vs baseline: 51.4500x; 51.4500x over previous
import functools

import jax
import jax.numpy as jnp
from jax import lax
from jax.experimental import pallas as pl
from jax.experimental.pallas import tpu as pltpu
from jax.experimental.pallas import tpu_sc as plsc

NC = 2
NS = 16
NW = NC * NS
LANES = 16
KCH = 64


def _scan_kernel(n_pad, cap, capp, with_deg):
    nr = n_pad // 128
    stripe = 8
    nstripe_tiles = nr // stripe
    steps = cap // LANES

    def body(*refs):
        if with_deg:
            (src_h, dst_h, mask_h, lsrc_h, ldst_h, cnt_h, msrc_h, deg_h,
             src_v, dst_v, mask_v, lsrc_v, ldst_v, marker_v, deg_v,
             idx_v, cnt16_v, zrow_v, sh_msrc, sh_deg) = refs
        else:
            (src_h, dst_h, mask_h, lsrc_h, ldst_h, cnt_h, msrc_h,
             src_v, dst_v, mask_v, lsrc_v, ldst_v, marker_v,
             idx_v, cnt16_v, zrow_v, sh_msrc) = refs
        cid = lax.axis_index("c")
        sid = lax.axis_index("s")
        wid = cid * NS + sid

        pltpu.sync_copy(src_h.at[wid], src_v)
        pltpu.sync_copy(dst_h.at[wid], dst_v)
        pltpu.sync_copy(mask_h, mask_v)

        zf = jnp.zeros((LANES,), jnp.float32)

        def zero_marker(i, _):
            r = i // 8
            c8 = i % 8
            marker_v[r, pl.ds(c8 * LANES, LANES)] = zf
            if with_deg:
                deg_v[r, pl.ds(c8 * LANES, LANES)] = zf
            return 0

        lax.fori_loop(0, nr * 8, zero_marker, 0)

        for j in range(nr // LANES):
            idx_v[0, pl.ds(j * LANES, LANES)] = (
                lax.iota(jnp.int32, LANES) + j * LANES)
        for r in range(stripe):
            for c8 in range(8):
                zrow_v[r, pl.ds(c8 * LANES, LANES)] = zf

        @pl.when(sid < nstripe_tiles)
        def _():
            pltpu.sync_copy(zrow_v, sh_msrc.at[pl.ds(sid * stripe, stripe)])
            if with_deg:
                pltpu.sync_copy(zrow_v, sh_deg.at[pl.ds(sid * stripe, stripe)])
        plsc.subcore_barrier()

        ones = jnp.ones((LANES,), jnp.float32)

        def step(i, cnt):
            b = i * LANES
            dv = dst_v[pl.ds(b, LANES)]
            sv = src_v[pl.ds(b, LANES)]
            mv = plsc.load_gather(mask_v, [dv])
            m = mv != 0.0
            plsc.store_compressed(lsrc_v.at[pl.ds(cnt, LANES)], sv, mask=m)
            plsc.store_compressed(ldst_v.at[pl.ds(cnt, LANES)], dv, mask=m)
            plsc.addupdate_scatter(
                marker_v,
                [lax.shift_right_logical(sv, 7), lax.bitwise_and(sv, 127)],
                ones, mask=m)
            if with_deg:
                plsc.addupdate_scatter(
                    deg_v,
                    [lax.shift_right_logical(dv, 7), lax.bitwise_and(dv, 127)],
                    ones)
            return cnt + jnp.sum(m.astype(jnp.int32))

        cnt = lax.fori_loop(0, steps, step, jnp.int32(0))

        padval = jnp.zeros((LANES,), jnp.int32) + (n_pad - NW + wid)
        for j in range(KCH // LANES):
            lsrc_v[pl.ds(cnt + j * LANES, LANES)] = padval
            ldst_v[pl.ds(cnt + j * LANES, LANES)] = padval

        cnt16_v[...] = jnp.zeros((LANES,), jnp.int32) + cnt
        pltpu.sync_copy(cnt16_v, cnt_h.at[wid])
        pltpu.sync_copy(lsrc_v, lsrc_h.at[wid])
        pltpu.sync_copy(ldst_v, ldst_h.at[wid])

        pltpu.sync_copy(marker_v, sh_msrc.at[idx_v.at[0]], add=True)
        if with_deg:
            pltpu.sync_copy(deg_v, sh_deg.at[idx_v.at[0]], add=True)
        plsc.subcore_barrier()

        @pl.when(sid < nstripe_tiles)
        def _():
            r0 = sid * stripe
            pltpu.sync_copy(sh_msrc.at[pl.ds(r0, stripe)],
                            msrc_h.at[pl.ds(cid * nr + r0, stripe)])
            if with_deg:
                pltpu.sync_copy(sh_deg.at[pl.ds(r0, stripe)],
                                deg_h.at[pl.ds(cid * nr + r0, stripe)])

    nr = n_pad // 128
    out_type = [
        jax.ShapeDtypeStruct((NW, capp), jnp.int32),
        jax.ShapeDtypeStruct((NW, capp), jnp.int32),
        jax.ShapeDtypeStruct((NW, LANES), jnp.int32),
        jax.ShapeDtypeStruct((NC * nr, 128), jnp.float32),
    ]
    scratch = [
        pltpu.VMEM((cap,), jnp.int32),
        pltpu.VMEM((cap,), jnp.int32),
        pltpu.VMEM((n_pad,), jnp.float32),
        pltpu.VMEM((capp,), jnp.int32),
        pltpu.VMEM((capp,), jnp.int32),
        pltpu.VMEM((nr, 128), jnp.float32),
    ]
    if with_deg:
        out_type.append(jax.ShapeDtypeStruct((NC * nr, 128), jnp.float32))
        scratch.append(pltpu.VMEM((nr, 128), jnp.float32))
    scratch += [
        pltpu.VMEM((1, nr), jnp.int32),
        pltpu.VMEM((LANES,), jnp.int32),
        pltpu.VMEM((8, 128), jnp.float32),
        pltpu.VMEM_SHARED((nr, 128), jnp.float32),
    ]
    if with_deg:
        scratch.append(pltpu.VMEM_SHARED((nr, 128), jnp.float32))

    mesh = plsc.VectorSubcoreMesh(core_axis_name="c", subcore_axis_name="s")
    return pl.kernel(body, out_type=tuple(out_type), mesh=mesh,
                     scratch_types=tuple(scratch),
                     compiler_params=pltpu.CompilerParams(
                         needs_layout_passes=False),
                     name=f"gnn_scan_deg{int(with_deg)}")


def _agg_kernel(n_pad, capp):
    nch = capp // KCH
    rows_per_tile = n_pad // NS
    blocks_per_tile = rows_per_tile // KCH

    def body(lsrc_h, ldst2_h, cnt_h, xs_h, out_h,
             lsrc_v, ldst2_v, cnt16_v, rows_v, zrow_v, sh_agg, sem):
        cid = lax.axis_index("c")
        sid = lax.axis_index("s")
        wid = cid * NS + sid

        pltpu.sync_copy(lsrc_h.at[wid], lsrc_v)
        pltpu.sync_copy(ldst2_h.at[wid], ldst2_v)
        pltpu.sync_copy(cnt_h.at[wid], cnt16_v)

        zf = jnp.zeros((LANES,), jnp.float32)

        def zero_row(i, _):
            r = i // 8
            c8 = i % 8
            zrow_v[r, pl.ds(c8 * LANES, LANES)] = zf
            return 0

        lax.fori_loop(0, KCH * 8, zero_row, 0)
        for b in range(blocks_per_tile):
            pltpu.sync_copy(
                zrow_v, sh_agg.at[pl.ds(sid * rows_per_tile + b * KCH, KCH)])
        plsc.subcore_barrier()

        cnt = jnp.max(cnt16_v[...])
        nch_live = (cnt + (KCH - 1)) // KCH

        def chunk(c, _):
            pltpu.async_copy(
                xs_h.at[lsrc_v.at[pl.ds(c * KCH, KCH)]], rows_v, sem).wait()
            pltpu.sync_copy(rows_v, sh_agg.at[ldst2_v.at[c]], add=True)
            return 0

        lax.fori_loop(0, nch_live, chunk, 0)
        plsc.subcore_barrier()

        for b in range(blocks_per_tile):
            r0 = sid * rows_per_tile + b * KCH
            pltpu.sync_copy(sh_agg.at[pl.ds(r0, KCH)],
                            out_h.at[pl.ds(cid * n_pad + r0, KCH)])

    mesh = plsc.VectorSubcoreMesh(core_axis_name="c", subcore_axis_name="s")
    return pl.kernel(
        body,
        out_type=jax.ShapeDtypeStruct((NC * n_pad, 128), jnp.float32),
        mesh=mesh,
        compiler_params=pltpu.CompilerParams(needs_layout_passes=False),
        scratch_types=(
            pltpu.VMEM((capp,), jnp.int32),
            pltpu.VMEM((nch, KCH), jnp.int32),
            pltpu.VMEM((LANES,), jnp.int32),
            pltpu.VMEM((KCH, 128), jnp.float32),
            pltpu.VMEM((KCH, 128), jnp.float32),
            pltpu.VMEM_SHARED((n_pad, 128), jnp.float32),
            pltpu.SemaphoreType.DMA,
        ),
        name="gnn_agg")


def _prep_body(degp, msp, oneh, dinv_o, mask1_o):
    nr = oneh.shape[0]
    deg = degp[:nr, :] + degp[nr:, :] + 1.0
    dinv_o[...] = lax.rsqrt(deg)
    mask1_o[...] = msp[:nr, :] + msp[nr:, :] + oneh[...]


def _mask_merge_body(msp, mprev, mask_o):
    nr = mprev.shape[0]
    mask_o[...] = msp[:nr, :] + msp[nr:, :] + mprev[...]


def _scale_body(x, dbc, o):
    o[...] = dbc[...] * x[...]


def _layer_body(sa, sb, xsp, dbc, w, b, o):
    agg = dbc[...] * (sa[...] + sb[...] + xsp[...])
    h = lax.dot_general(agg, w[...], (((1,), (0,)), ((), ())),
                        precision=lax.Precision.HIGHEST,
                        preferred_element_type=jnp.float32)
    o[...] = dbc[...] * jnp.maximum(h + b[...], 0.0)


def _final_body(sa, sb, xst, dt, w3, b3, wfc, bfc, o):
    agg = dt[...] * (sa[...] + sb[...] + xst[...])
    h = lax.dot_general(agg, w3[...], (((1,), (0,)), ((), ())),
                        precision=lax.Precision.HIGHEST,
                        preferred_element_type=jnp.float32)
    h = jnp.maximum(h + b3[...], 0.0)
    o[...] = lax.dot_general(h, wfc[...], (((1,), (0,)), ((), ())),
                             precision=lax.Precision.HIGHEST,
                             preferred_element_type=jnp.float32) + bfc[...]


def _layer_call(n_pad, name):
    nblk = 16
    blk = n_pad // nblk
    rows = pl.BlockSpec((blk, 128), lambda i: (i, 0))
    rows_hi = pl.BlockSpec((blk, 128), lambda i: (i + nblk, 0))
    full = pl.BlockSpec((128, 128), lambda i: (0, 0))
    vec = pl.BlockSpec((1, 128), lambda i: (0, 0))
    return pl.pallas_call(
        _layer_body,
        grid=(nblk,),
        in_specs=[rows, rows_hi, rows, rows, full, vec],
        out_specs=rows,
        out_shape=jax.ShapeDtypeStruct((n_pad, 128), jnp.float32),
        name=name)


@jax.jit
def kernel(x, edge_index, W1, b1, W2, b2, W3, b3, Wfc, bfc):
    n = x.shape[0]
    e = edge_index.shape[1]
    t = n - 1
    n_pad = ((n + 127) // 128 + (-((n + 127) // 128)) % NS) * 128
    while n_pad - NW < n:
        n_pad += NS * 128
    nr = n_pad // 128
    cap = e // NW
    capp = ((cap + KCH) + (KCH - 1)) // KCH * KCH

    src = edge_index[0].reshape(NW, cap)
    dst = edge_index[1].reshape(NW, cap)
    x_pad = jnp.pad(x, ((0, n_pad - n), (0, 0)))
    oneh_flat = (jnp.arange(n_pad) == t).astype(jnp.float32)
    oneh2d = oneh_flat.reshape(nr, 128)

    scan_deg = _scan_kernel(n_pad, cap, capp, with_deg=True)
    scan = _scan_kernel(n_pad, cap, capp, with_deg=False)
    agg = _agg_kernel(n_pad, capp)

    lsrc3, ldst3, cnt3, ms3, degp = scan_deg(src, dst, oneh_flat)

    whole = pl.BlockSpec((2 * nr, 128), lambda: (0, 0))
    plane = pl.BlockSpec((nr, 128), lambda: (0, 0))
    dinv2d, mask1_2d = pl.pallas_call(
        _prep_body,
        in_specs=[whole, whole, plane],
        out_specs=[plane, plane],
        out_shape=[jax.ShapeDtypeStruct((nr, 128), jnp.float32),
                   jax.ShapeDtypeStruct((nr, 128), jnp.float32)],
        name="gnn_prep")(degp, ms3, oneh2d)

    dinv_bc = jnp.broadcast_to(dinv2d.reshape(n_pad, 1), (n_pad, 128))

    lsrc2, ldst2, cnt2, ms2 = scan(src, dst, mask1_2d.reshape(n_pad))
    mask2_2d = pl.pallas_call(
        _mask_merge_body,
        in_specs=[whole, plane],
        out_specs=plane,
        out_shape=jax.ShapeDtypeStruct((nr, 128), jnp.float32),
        name="gnn_mask_merge")(ms2, mask1_2d)

    lsrc1, ldst1, cnt1, _ = scan(src, dst, mask2_2d.reshape(n_pad))

    nblk = 16
    blk = n_pad // nblk
    rows = pl.BlockSpec((blk, 128), lambda i: (i, 0))
    xs0 = pl.pallas_call(
        _scale_body, grid=(nblk,),
        in_specs=[rows, rows], out_specs=rows,
        out_shape=jax.ShapeDtypeStruct((n_pad, 128), jnp.float32),
        name="gnn_scale")(x_pad, dinv_bc)

    def as2d(l):
        return l.reshape(NW, capp // KCH, KCH)

    s1 = agg(lsrc1, as2d(ldst1), cnt1, xs0)
    xs1 = _layer_call(n_pad, "gnn_layer1")(s1, s1, xs0, dinv_bc, W1,
                                           b1.reshape(1, 128))
    s2 = agg(lsrc2, as2d(ldst2), cnt2, xs1)
    xs2 = _layer_call(n_pad, "gnn_layer2")(s2, s2, xs1, dinv_bc, W2,
                                           b2.reshape(1, 128))
    s3 = agg(lsrc3, as2d(ldst3), cnt3, xs2)

    row = pl.BlockSpec((1, 128), lambda: (0, 0))
    mat = pl.BlockSpec((128, 128), lambda: (0, 0))
    out = pl.pallas_call(
        _final_body,
        in_specs=[row, row, row, row, mat, row, mat, row],
        out_specs=row,
        out_shape=jax.ShapeDtypeStruct((1, 128), jnp.float32),
        name="gnn_final")(
            lax.dynamic_slice(s3, (t, 0), (1, 128)),
            lax.dynamic_slice(s3, (n_pad + t, 0), (1, 128)),
            lax.dynamic_slice(xs2, (t, 0), (1, 128)),
            lax.dynamic_slice(dinv_bc, (t, 0), (1, 128)),
            W3, b3.reshape(1, 128), Wfc, bfc.reshape(1, 128))
    return out.reshape(128)
